# Initial kernel scaffold; baseline (speedup 1.0000x reference)
import functools

import jax
import jax.numpy as jnp
from jax import lax
from jax.experimental import pallas as pl
from jax.experimental.pallas import tpu as pltpu
from jax.experimental.pallas import tpu_sc as plsc

N_NODES = 10000
N_EDGES = 320000
D = 128
NPAD = 10240
NC = 2
NS = 16
NW = NC * NS

EPT_A = N_EDGES // NS
EPT_C = N_EDGES // NW
CHUNK = 128

_MESH = plsc.VectorSubcoreMesh(core_axis_name="c", subcore_axis_name="s")


def _newton_rsqrt(d):
  bits = plsc.bitcast(d, jnp.int32)
  y = plsc.bitcast(jnp.int32(0x5F3759DF) - (bits >> 1), jnp.float32)
  for _ in range(3):
    y = y * (1.5 - 0.5 * d * y * y)
  return y


def _deg_dis_body(dst_hbm, zs_hbm, dis_hbm, deg_sp, idx_v, ones_v, seg_v):
  cid = lax.axis_index("c")
  sid = lax.axis_index("s")

  @pl.when(cid == 0)
  def _():
    pltpu.sync_copy(zs_hbm.at[pl.ds(sid * 640, 640)],
                    deg_sp.at[pl.ds(sid * 640, 640)])
    for i in range(8):
      ones_v[pl.ds(i * 16, 16)] = jnp.full((16,), 1.0, jnp.float32)
    plsc.subcore_barrier()

    base = sid * EPT_A

    def body(j, carry):
      pltpu.sync_copy(dst_hbm.at[pl.ds(base + j * CHUNK, CHUNK)],
                      idx_v.at[0])
      pltpu.sync_copy(ones_v, deg_sp.at[idx_v.at[0]], add=True)
      return carry

    lax.fori_loop(0, EPT_A // CHUNK, body, 0)
    rem = EPT_A % CHUNK
    pltpu.sync_copy(dst_hbm.at[pl.ds(base + (EPT_A // CHUNK) * CHUNK, rem)],
                    idx_v.at[1, pl.ds(0, rem)])
    pltpu.sync_copy(ones_v.at[pl.ds(0, rem)],
                    deg_sp.at[idx_v.at[1, pl.ds(0, rem)]], add=True)

    plsc.subcore_barrier()
    pltpu.sync_copy(deg_sp.at[pl.ds(sid * 640, 640)], seg_v)
    for i in range(40):
      d = seg_v[pl.ds(i * 16, 16)] + 1.0
      seg_v[pl.ds(i * 16, 16)] = _newton_rsqrt(d)
    pltpu.sync_copy(seg_v, dis_hbm.at[pl.ds(sid * 640, 640)])


_deg_dis = pl.kernel(
    _deg_dis_body,
    out_type=jax.ShapeDtypeStruct((NPAD,), jnp.float32),
    mesh=_MESH,
    scratch_types=[
        pltpu.VMEM_SHARED((NPAD,), jnp.float32),
        pltpu.VMEM((2, CHUNK), jnp.int32),
        pltpu.VMEM((CHUNK,), jnp.float32),
        pltpu.VMEM((640,), jnp.float32),
    ],
)


def _proj_body(x_ref, w_ref, dis_ref, g_ref):
  h = jnp.dot(x_ref[...], w_ref[...], preferred_element_type=jnp.float32)
  g_ref[...] = dis_ref[...] * h


def _proj(x, W1, dis_col):
  bm = 2000
  return pl.pallas_call(
      _proj_body,
      grid=(N_NODES // bm,),
      in_specs=[
          pl.BlockSpec((bm, D), lambda i: (i, 0)),
          pl.BlockSpec((D, D), lambda i: (0, 0)),
          pl.BlockSpec((bm, 1), lambda i: (i, 0)),
      ],
      out_specs=pl.BlockSpec((bm, D), lambda i: (i, 0)),
      out_shape=jax.ShapeDtypeStruct((N_NODES, D), jnp.float32),
  )(x, W1, dis_col)


def _edge_body(src_hbm, dst_hbm, dis_hbm, g_hbm, zrow_hbm, zs_hbm,
               sp_hbm, accp_hbm,
               acc_sp, s_sp, dis_v, sidx, didx, val_v, row_v, sem):
  cid = lax.axis_index("c")
  sid = lax.axis_index("s")
  wid = cid * NS + sid
  base = wid * EPT_C

  pltpu.sync_copy(zrow_hbm.at[pl.ds(sid * 625, 625)],
                  acc_sp.at[pl.ds(sid * 625, 625)])
  pltpu.sync_copy(zs_hbm.at[pl.ds(sid * 640, 640)],
                  s_sp.at[pl.ds(sid * 640, 640)])
  pltpu.sync_copy(dis_hbm, dis_v)
  plsc.subcore_barrier()

  nch = EPT_C // CHUNK

  def s_body(j, carry):
    pltpu.sync_copy(dst_hbm.at[pl.ds(base + j * CHUNK, CHUNK)], didx.at[0])
    pltpu.sync_copy(src_hbm.at[pl.ds(base + j * CHUNK, CHUNK)], sidx.at[0])
    for k in range(8):
      dv = didx[0, pl.ds(k * 16, 16)]
      val_v[pl.ds(k * 16, 16)] = plsc.load_gather(dis_v, [dv])
    pltpu.sync_copy(val_v, s_sp.at[sidx.at[0]], add=True)
    return carry

  lax.fori_loop(0, nch, s_body, 0)
  rem = EPT_C % CHUNK
  rbase = base + nch * CHUNK
  pltpu.sync_copy(dst_hbm.at[pl.ds(rbase, rem)], didx.at[1, pl.ds(0, rem)])
  pltpu.sync_copy(src_hbm.at[pl.ds(rbase, rem)], sidx.at[1, pl.ds(0, rem)])
  dv = didx[1, pl.ds(0, 16)]
  val_v[pl.ds(0, 16)] = plsc.load_gather(dis_v, [dv])
  pltpu.sync_copy(val_v.at[pl.ds(0, rem)],
                  s_sp.at[sidx.at[1, pl.ds(0, rem)]], add=True)

  def acc_body(j, carry):
    pltpu.sync_copy(src_hbm.at[pl.ds(base + j * CHUNK, CHUNK)], sidx.at[0])
    pltpu.sync_copy(dst_hbm.at[pl.ds(base + j * CHUNK, CHUNK)], didx.at[0])
    pltpu.async_copy(g_hbm.at[sidx.at[0]], row_v, sem).wait()
    pltpu.sync_copy(row_v, acc_sp.at[didx.at[0]], add=True)
    return carry

  lax.fori_loop(0, nch, acc_body, 0)
  pltpu.sync_copy(src_hbm.at[pl.ds(rbase, rem)], sidx.at[1, pl.ds(0, rem)])
  pltpu.sync_copy(dst_hbm.at[pl.ds(rbase, rem)], didx.at[1, pl.ds(0, rem)])
  pltpu.async_copy(g_hbm.at[sidx.at[1, pl.ds(0, rem)]],
                   row_v.at[pl.ds(0, rem)], sem).wait()
  pltpu.sync_copy(row_v.at[pl.ds(0, rem)],
                  acc_sp.at[didx.at[1, pl.ds(0, rem)]], add=True)

  plsc.subcore_barrier()
  pltpu.sync_copy(acc_sp.at[pl.ds(sid * 625, 625)],
                  accp_hbm.at[cid, pl.ds(sid * 625, 625)])
  pltpu.sync_copy(s_sp.at[pl.ds(sid * 640, 640)],
                  sp_hbm.at[cid, pl.ds(sid * 640, 640)])


_edge_pass = pl.kernel(
    _edge_body,
    out_type=(
        jax.ShapeDtypeStruct((NC, NPAD), jnp.float32),
        jax.ShapeDtypeStruct((NC, N_NODES, D), jnp.float32),
    ),
    mesh=_MESH,
    scratch_types=[
        pltpu.VMEM_SHARED((N_NODES, D), jnp.float32),
        pltpu.VMEM_SHARED((NPAD,), jnp.float32),
        pltpu.VMEM((NPAD,), jnp.float32),
        pltpu.VMEM((2, CHUNK), jnp.int32),
        pltpu.VMEM((2, CHUNK), jnp.int32),
        pltpu.VMEM((CHUNK,), jnp.float32),
        pltpu.VMEM((CHUNK, D), jnp.float32),
        pltpu.SemaphoreType.DMA,
    ],
)


def _head_body(dis_ref, g_ref, accp_ref, sp_ref, b1_ref, w2_ref, b2_ref,
               wfc_ref, bfc_ref, out_ref, vacc):
  i = pl.program_id(0)
  dis = dis_ref[...]
  acc = accp_ref[0] + accp_ref[1]
  z1 = dis * (acc + g_ref[...]) + b1_ref[...]
  r1 = jnp.maximum(z1, 0.0)
  s = sp_ref[0] + sp_ref[1]
  w = dis * (dis + s)
  contrib = jnp.sum(w * r1, axis=0, keepdims=True)

  @pl.when(i == 0)
  def _():
    vacc[...] = jnp.zeros_like(vacc)

  vacc[...] += contrib

  @pl.when(i == pl.num_programs(0) - 1)
  def _():
    pooled = jnp.dot(vacc[...] * (1.0 / N_NODES), w2_ref[...],
                     preferred_element_type=jnp.float32) + b2_ref[...]
    out_ref[...] = jnp.dot(pooled, wfc_ref[...],
                           preferred_element_type=jnp.float32) + bfc_ref[...]


def _head(dis_col, g, accp, sp, b1, W2, b2, Wfc, bfc):
  bm = 2000
  return pl.pallas_call(
      _head_body,
      grid=(N_NODES // bm,),
      in_specs=[
          pl.BlockSpec((bm, 1), lambda i: (i, 0)),
          pl.BlockSpec((bm, D), lambda i: (i, 0)),
          pl.BlockSpec((NC, bm, D), lambda i: (0, i, 0)),
          pl.BlockSpec((NC, bm, 1), lambda i: (0, i, 0)),
          pl.BlockSpec((1, D), lambda i: (0, 0)),
          pl.BlockSpec((D, D), lambda i: (0, 0)),
          pl.BlockSpec((1, D), lambda i: (0, 0)),
          pl.BlockSpec((D, D), lambda i: (0, 0)),
          pl.BlockSpec((1, D), lambda i: (0, 0)),
      ],
      out_specs=pl.BlockSpec((1, D), lambda i: (0, 0)),
      out_shape=jax.ShapeDtypeStruct((1, D), jnp.float32),
      scratch_shapes=[pltpu.VMEM((1, D), jnp.float32)],
  )(dis_col, g, accp, sp, b1, W2, b2, Wfc, bfc)


def kernel(x, edge_index, W1, b1, W2, b2, Wfc, bfc):
  src = edge_index[0].astype(jnp.int32)
  dst = edge_index[1].astype(jnp.int32)
  zrow = jnp.zeros((N_NODES, D), jnp.float32)
  zs = jnp.zeros((NPAD,), jnp.float32)

  dis = _deg_dis(dst, zs)
  dis_col = dis[:N_NODES].reshape(N_NODES, 1)
  g = _proj(x, W1, dis_col)
  sp, accp = _edge_pass(src, dst, dis, g, zrow, zs)
  sp_col = sp[:, :N_NODES].reshape(NC, N_NODES, 1)
  b1r = b1.reshape(1, D)
  b2r = b2.reshape(1, D)
  bfcr = bfc.reshape(1, D)
  return _head(dis_col, g, accp, sp_col, b1r, W2, b2r, Wfc, bfcr)


# speedup vs baseline: 20.2385x; 20.2385x over previous
import functools

import jax
import jax.numpy as jnp
from jax import lax
from jax.experimental import pallas as pl
from jax.experimental.pallas import tpu as pltpu
from jax.experimental.pallas import tpu_sc as plsc

N_NODES = 10000
N_EDGES = 320000
D = 128
NPAD = 10240
NC = 2
NS = 16
NW = NC * NS

EPT_A = N_EDGES // NS
EPT_C = N_EDGES // NW
CHUNK = 128

_MESH = plsc.VectorSubcoreMesh(core_axis_name="c", subcore_axis_name="s")


def _deg_body(dst_hbm, zs_hbm, deg_hbm, deg_sp, idx_v, ones_v):
  cid = lax.axis_index("c")
  sid = lax.axis_index("s")

  @pl.when(cid == 0)
  def _():
    pltpu.sync_copy(zs_hbm.at[pl.ds(sid * 640, 640)],
                    deg_sp.at[pl.ds(sid * 640, 640)])
    for i in range(8):
      ones_v[pl.ds(i * 16, 16)] = jnp.full((16,), 1.0, jnp.float32)
    plsc.subcore_barrier()

    base = sid * EPT_A

    def body(j, carry):
      pltpu.sync_copy(dst_hbm.at[pl.ds(base + j * CHUNK, CHUNK)],
                      idx_v.at[0])
      pltpu.sync_copy(ones_v, deg_sp.at[idx_v.at[0]], add=True)
      return carry

    lax.fori_loop(0, EPT_A // CHUNK, body, 0)
    rem = EPT_A % CHUNK
    pltpu.sync_copy(dst_hbm.at[pl.ds(base + (EPT_A // CHUNK) * CHUNK, rem)],
                    idx_v.at[1, pl.ds(0, rem)])
    pltpu.sync_copy(ones_v.at[pl.ds(0, rem)],
                    deg_sp.at[idx_v.at[1, pl.ds(0, rem)]], add=True)

    plsc.subcore_barrier()
    pltpu.sync_copy(deg_sp.at[pl.ds(sid * 640, 640)],
                    deg_hbm.at[pl.ds(sid * 640, 640)])


_deg = pl.kernel(
    _deg_body,
    out_type=jax.ShapeDtypeStruct((NPAD,), jnp.float32),
    mesh=_MESH,
    compiler_params=pltpu.CompilerParams(needs_layout_passes=False),
    scratch_types=[
        pltpu.VMEM_SHARED((NPAD,), jnp.float32),
        pltpu.VMEM((2, CHUNK), jnp.int32),
        pltpu.VMEM((CHUNK,), jnp.float32),
    ],
)


def _proj_body(x_ref, w_ref, deg_ref, g_ref, dis_ref):
  dis = lax.rsqrt(deg_ref[...] + 1.0)
  dis_ref[...] = dis
  h = jnp.dot(x_ref[...], w_ref[...], preferred_element_type=jnp.float32)
  g_ref[...] = dis * h


def _proj(x, W1, deg_col):
  bm = 2000
  return pl.pallas_call(
      _proj_body,
      grid=(N_NODES // bm,),
      in_specs=[
          pl.BlockSpec((bm, D), lambda i: (i, 0)),
          pl.BlockSpec((D, D), lambda i: (0, 0)),
          pl.BlockSpec((bm, 1), lambda i: (i, 0)),
      ],
      out_specs=[
          pl.BlockSpec((bm, D), lambda i: (i, 0)),
          pl.BlockSpec((bm, 1), lambda i: (i, 0)),
      ],
      out_shape=[
          jax.ShapeDtypeStruct((N_NODES, D), jnp.float32),
          jax.ShapeDtypeStruct((N_NODES, 1), jnp.float32),
      ],
  )(x, W1, deg_col)


def _edge_body(src_hbm, dst_hbm, dis_hbm, g_hbm, zrow_hbm, zs_hbm,
               sp_hbm, accp_hbm,
               acc_sp, s_sp, dis_v, sidx, didx, val_v, row_v, sem):
  cid = lax.axis_index("c")
  sid = lax.axis_index("s")
  wid = cid * NS + sid
  base = wid * EPT_C

  pltpu.sync_copy(zrow_hbm.at[pl.ds(sid * 640, 640)],
                  acc_sp.at[pl.ds(sid * 640, 640)])
  pltpu.sync_copy(zs_hbm.at[pl.ds(sid * 640, 640)],
                  s_sp.at[pl.ds(sid * 640, 640)])
  pltpu.sync_copy(dis_hbm, dis_v)
  plsc.subcore_barrier()

  nch = EPT_C // CHUNK

  def s_body(j, carry):
    pltpu.sync_copy(dst_hbm.at[pl.ds(base + j * CHUNK, CHUNK)], didx.at[0])
    pltpu.sync_copy(src_hbm.at[pl.ds(base + j * CHUNK, CHUNK)], sidx.at[0])
    for k in range(8):
      dv = didx[0, pl.ds(k * 16, 16)]
      val_v[pl.ds(k * 16, 16)] = plsc.load_gather(dis_v, [dv])
    pltpu.sync_copy(val_v, s_sp.at[sidx.at[0]], add=True)
    return carry

  lax.fori_loop(0, nch, s_body, 0)
  rem = EPT_C % CHUNK
  rbase = base + nch * CHUNK
  pltpu.sync_copy(dst_hbm.at[pl.ds(rbase, rem)], didx.at[1, pl.ds(0, rem)])
  pltpu.sync_copy(src_hbm.at[pl.ds(rbase, rem)], sidx.at[1, pl.ds(0, rem)])
  dv = didx[1, pl.ds(0, 16)]
  val_v[pl.ds(0, 16)] = plsc.load_gather(dis_v, [dv])
  pltpu.sync_copy(val_v.at[pl.ds(0, rem)],
                  s_sp.at[sidx.at[1, pl.ds(0, rem)]], add=True)

  def acc_body(j, carry):
    pltpu.sync_copy(src_hbm.at[pl.ds(base + j * CHUNK, CHUNK)], sidx.at[0])
    pltpu.sync_copy(dst_hbm.at[pl.ds(base + j * CHUNK, CHUNK)], didx.at[0])
    pltpu.async_copy(g_hbm.at[sidx.at[0]], row_v, sem).wait()
    pltpu.sync_copy(row_v, acc_sp.at[didx.at[0]], add=True)
    return carry

  lax.fori_loop(0, nch, acc_body, 0)
  pltpu.sync_copy(src_hbm.at[pl.ds(rbase, rem)], sidx.at[1, pl.ds(0, rem)])
  pltpu.sync_copy(dst_hbm.at[pl.ds(rbase, rem)], didx.at[1, pl.ds(0, rem)])
  pltpu.async_copy(g_hbm.at[sidx.at[1, pl.ds(0, rem)]],
                   row_v.at[pl.ds(0, rem)], sem).wait()
  pltpu.sync_copy(row_v.at[pl.ds(0, rem)],
                  acc_sp.at[didx.at[1, pl.ds(0, rem)]], add=True)

  plsc.subcore_barrier()
  pltpu.sync_copy(acc_sp.at[pl.ds(sid * 640, 640)],
                  accp_hbm.at[cid, pl.ds(sid * 640, 640)])
  pltpu.sync_copy(s_sp.at[pl.ds(sid * 640, 640)],
                  sp_hbm.at[cid, pl.ds(sid * 640, 640)])


_edge_pass = pl.kernel(
    _edge_body,
    out_type=(
        jax.ShapeDtypeStruct((NC, NPAD), jnp.float32),
        jax.ShapeDtypeStruct((NC, NPAD, D), jnp.float32),
    ),
    mesh=_MESH,
    compiler_params=pltpu.CompilerParams(needs_layout_passes=False),
    scratch_types=[
        pltpu.VMEM_SHARED((NPAD, D), jnp.float32),
        pltpu.VMEM_SHARED((NPAD,), jnp.float32),
        pltpu.VMEM((N_NODES,), jnp.float32),
        pltpu.VMEM((2, CHUNK), jnp.int32),
        pltpu.VMEM((2, CHUNK), jnp.int32),
        pltpu.VMEM((CHUNK,), jnp.float32),
        pltpu.VMEM((CHUNK, D), jnp.float32),
        pltpu.SemaphoreType.DMA,
    ],
)


def _head_body(dis_ref, g_ref, accp_ref, sp_ref, b1_ref, w2_ref, b2_ref,
               wfc_ref, bfc_ref, out_ref, vacc):
  i = pl.program_id(0)
  dis = dis_ref[...]
  acc = accp_ref[0] + accp_ref[1]
  z1 = dis * (acc + g_ref[...]) + b1_ref[...]
  r1 = jnp.maximum(z1, 0.0)
  s = sp_ref[0] + sp_ref[1]
  w = dis * (dis + s)
  contrib = jnp.sum(w * r1, axis=0, keepdims=True)

  @pl.when(i == 0)
  def _():
    vacc[...] = jnp.zeros_like(vacc)

  vacc[...] += contrib

  @pl.when(i == pl.num_programs(0) - 1)
  def _():
    pooled = jnp.dot(vacc[...] * (1.0 / N_NODES), w2_ref[...],
                     preferred_element_type=jnp.float32) + b2_ref[...]
    out_ref[...] = jnp.dot(pooled, wfc_ref[...],
                           preferred_element_type=jnp.float32) + bfc_ref[...]


def _head(dis_col, g, accp, sp, b1, W2, b2, Wfc, bfc):
  bm = 2000
  return pl.pallas_call(
      _head_body,
      grid=(N_NODES // bm,),
      in_specs=[
          pl.BlockSpec((bm, 1), lambda i: (i, 0)),
          pl.BlockSpec((bm, D), lambda i: (i, 0)),
          pl.BlockSpec((NC, bm, D), lambda i: (0, i, 0)),
          pl.BlockSpec((NC, bm, 1), lambda i: (0, i, 0)),
          pl.BlockSpec((1, D), lambda i: (0, 0)),
          pl.BlockSpec((D, D), lambda i: (0, 0)),
          pl.BlockSpec((1, D), lambda i: (0, 0)),
          pl.BlockSpec((D, D), lambda i: (0, 0)),
          pl.BlockSpec((1, D), lambda i: (0, 0)),
      ],
      out_specs=pl.BlockSpec((1, D), lambda i: (0, 0)),
      out_shape=jax.ShapeDtypeStruct((1, D), jnp.float32),
      scratch_shapes=[pltpu.VMEM((1, D), jnp.float32)],
  )(dis_col, g, accp, sp, b1, W2, b2, Wfc, bfc)


def kernel(x, edge_index, W1, b1, W2, b2, Wfc, bfc):
  src = edge_index[0].astype(jnp.int32)
  dst = edge_index[1].astype(jnp.int32)
  zrow = jnp.zeros((NPAD, D), jnp.float32)
  zs = jnp.zeros((NPAD,), jnp.float32)

  deg = _deg(dst, zs)
  deg_col = deg.reshape(NPAD, 1)
  g, dis_col = _proj(x, W1, deg_col)
  dis_flat = dis_col.reshape(N_NODES)
  sp, accp = _edge_pass(src, dst, dis_flat, g, zrow, zs)
  sp_col = sp.reshape(NC, NPAD, 1)
  b1r = b1.reshape(1, D)
  b2r = b2.reshape(1, D)
  bfcr = bfc.reshape(1, D)
  return _head(dis_col, g, accp, sp_col, b1r, W2, b2r, Wfc, bfcr)
